# baseline (speedup 1.0000x reference)
import jax
import jax.numpy as jnp
from jax import lax
from jax.experimental import pallas as pl
from jax.experimental.pallas import tpu as pltpu
from jax.experimental.pallas import tpu_sc as plsc

NUM_CORES = 2
NUM_SUBCORES = 16
LANES = 16
NUM_WORKERS = NUM_CORES * NUM_SUBCORES

BATCH = 16384
EMBED_DIM = 32
B_PER_W = BATCH // NUM_WORKERS
IDX_CHUNK = 128
N_IDX_CHUNKS = B_PER_W // IDX_CHUNK
N_ROW_CHUNKS = B_PER_W // LANES


def _body(uid_hbm, pid_hbm, nid_hbm, utab_hbm, itab_hbm,
          pos_hbm, neg_hbm,
          uid_v, pid_v, nid_v, urows, prows, nrows, pos_v, neg_v, sem):
  wid = lax.axis_index("s") * NUM_CORES + lax.axis_index("c")
  base = wid * B_PER_W

  for k in range(N_IDX_CHUNKS):
    off = base + k * IDX_CHUNK
    pltpu.sync_copy(uid_hbm.at[pl.ds(off, IDX_CHUNK)], uid_v.at[k])
    pltpu.sync_copy(pid_hbm.at[pl.ds(off, IDX_CHUNK)], pid_v.at[k])
    pltpu.sync_copy(nid_hbm.at[pl.ds(off, IDX_CHUNK)], nid_v.at[k])

  copies = []
  for k in range(N_IDX_CHUNKS):
    dst = pl.ds(k * IDX_CHUNK, IDX_CHUNK)
    copies.append(pltpu.async_copy(utab_hbm.at[uid_v.at[k]], urows.at[dst], sem))
    copies.append(pltpu.async_copy(itab_hbm.at[pid_v.at[k]], prows.at[dst], sem))
    copies.append(pltpu.async_copy(itab_hbm.at[nid_v.at[k]], nrows.at[dst], sem))
  for c in copies:
    c.wait()


  lane = lax.iota(jnp.int32, LANES)

  def chunk(j, carry):
    row_ids = j * LANES + lane
    accp = jnp.zeros((LANES,), jnp.float32)
    accn = jnp.zeros((LANES,), jnp.float32)
    for d in range(EMBED_DIM):
      cols = jnp.full((LANES,), d, jnp.int32)
      u = plsc.load_gather(urows, [row_ids, cols])
      p = plsc.load_gather(prows, [row_ids, cols])
      n = plsc.load_gather(nrows, [row_ids, cols])
      accp = accp + u * p
      accn = accn + u * n
    pos_v[pl.ds(j * LANES, LANES)] = accp
    neg_v[pl.ds(j * LANES, LANES)] = accn
    return carry

  lax.fori_loop(0, N_ROW_CHUNKS, chunk, 0)

  pltpu.sync_copy(pos_v, pos_hbm.at[pl.ds(base, B_PER_W)])
  pltpu.sync_copy(neg_v, neg_hbm.at[pl.ds(base, B_PER_W)])


@jax.jit
def kernel(user_ids, pos_item_ids, neg_item_ids, user_table, item_table):
  user_ids = user_ids.astype(jnp.int32)
  pos_item_ids = pos_item_ids.astype(jnp.int32)
  neg_item_ids = neg_item_ids.astype(jnp.int32)

  mesh = plsc.VectorSubcoreMesh(core_axis_name="c", subcore_axis_name="s")
  f = pl.kernel(
      _body,
      out_type=(
          jax.ShapeDtypeStruct((BATCH,), jnp.float32),
          jax.ShapeDtypeStruct((BATCH,), jnp.float32),
      ),
      mesh=mesh,
      scratch_types=(
          pltpu.VMEM((N_IDX_CHUNKS, IDX_CHUNK), jnp.int32),
          pltpu.VMEM((N_IDX_CHUNKS, IDX_CHUNK), jnp.int32),
          pltpu.VMEM((N_IDX_CHUNKS, IDX_CHUNK), jnp.int32),
          pltpu.VMEM((B_PER_W, EMBED_DIM), jnp.float32),
          pltpu.VMEM((B_PER_W, EMBED_DIM), jnp.float32),
          pltpu.VMEM((B_PER_W, EMBED_DIM), jnp.float32),
          pltpu.VMEM((B_PER_W,), jnp.float32),
          pltpu.VMEM((B_PER_W,), jnp.float32),
          pltpu.SemaphoreType.DMA,
      ),
      compiler_params=pltpu.CompilerParams(
          needs_layout_passes=False, use_tc_tiling_on_sc=False),
  )
  return f(user_ids, pos_item_ids, neg_item_ids, user_table, item_table)
